# Initial kernel scaffold; baseline (speedup 1.0000x reference)
import jax
import jax.numpy as jnp
from jax.experimental import pallas as pl


def kernel(x, W, b):
    raise NotImplementedError("write your pallas kernel here")


# speedup vs baseline: 1.1685x; 1.1685x over previous
import functools

import jax
import jax.numpy as jnp
from jax.experimental import pallas as pl

D_MODEL = 4096
NUM_EXPERTS = 64
TOP_K = 8
BT = 512


def _router_kernel(x_ref, wt_ref, b_ref, idx_ref, p_ref):
    logits = jax.lax.dot_general(
        x_ref[...], wt_ref[...],
        dimension_numbers=(((1,), (0,)), ((), ())),
        preferred_element_type=jnp.float32,
    ) + b_ref[...]
    m = jnp.max(logits, axis=-1, keepdims=True)
    e = jnp.exp(logits - m)
    probs = e / jnp.sum(e, axis=-1, keepdims=True)

    lane = jax.lax.broadcasted_iota(jnp.int32, probs.shape, 1)
    vals = []
    idxs = []
    work = probs
    for _ in range(TOP_K):
        v = jnp.max(work, axis=-1, keepdims=True)
        i = jnp.min(jnp.where(work == v, lane, NUM_EXPERTS), axis=-1,
                    keepdims=True)
        vals.append(v)
        idxs.append(i)
        work = jnp.where(lane == i, -1.0, work)
    topv = jnp.concatenate(vals, axis=-1)
    topi = jnp.concatenate(idxs, axis=-1)
    topv = topv / (jnp.sum(topv, axis=-1, keepdims=True) + 1e-08)
    idx_ref[...] = topi
    p_ref[...] = topv


@jax.jit
def kernel(x, W, b):
    n = x.shape[0]
    grid = (n // BT,)
    out_shape = (
        jax.ShapeDtypeStruct((n, TOP_K), jnp.int32),
        jax.ShapeDtypeStruct((n, TOP_K), jnp.float32),
    )
    topi, topv = pl.pallas_call(
        _router_kernel,
        grid=grid,
        in_specs=[
            pl.BlockSpec((BT, D_MODEL), lambda i: (i, 0)),
            pl.BlockSpec((D_MODEL, NUM_EXPERTS), lambda i: (0, 0)),
            pl.BlockSpec((1, NUM_EXPERTS), lambda i: (0, 0)),
        ],
        out_specs=(
            pl.BlockSpec((BT, TOP_K), lambda i: (i, 0)),
            pl.BlockSpec((BT, TOP_K), lambda i: (i, 0)),
        ),
        out_shape=out_shape,
    )(x, W.T, b.reshape(1, NUM_EXPERTS))
    return topi, topv


# speedup vs baseline: 1.7362x; 1.4859x over previous
import jax
import jax.numpy as jnp
from jax.experimental import pallas as pl

D_MODEL = 4096
NUM_EXPERTS = 64
TOP_K = 8
BT = 512


def _router_kernel(x_ref, w_ref, b_ref, idx_ref, p_ref):
    logits = jax.lax.dot_general(
        w_ref[...], x_ref[...],
        dimension_numbers=(((1,), (1,)), ((), ())),
        preferred_element_type=jnp.float32,
    ) + b_ref[...]
    m = jnp.max(logits, axis=0, keepdims=True)
    e = jnp.exp(logits - m)
    probs = e / jnp.sum(e, axis=0, keepdims=True)

    row = jax.lax.broadcasted_iota(jnp.int32, probs.shape, 0)
    vals = []
    idxs = []
    work = probs
    for _ in range(TOP_K):
        v = jnp.max(work, axis=0, keepdims=True)
        i = jnp.min(jnp.where(work == v, row, NUM_EXPERTS), axis=0,
                    keepdims=True)
        vals.append(v)
        idxs.append(i)
        work = jnp.where(row == i, -1.0, work)
    topv = jnp.concatenate(vals, axis=0)
    topi = jnp.concatenate(idxs, axis=0)
    topv = topv / (jnp.sum(topv, axis=0, keepdims=True) + 1e-08)
    idx_ref[...] = topi
    p_ref[...] = topv


@jax.jit
def kernel(x, W, b):
    n = x.shape[0]
    grid = (n // BT,)
    out_shape = (
        jax.ShapeDtypeStruct((TOP_K, n), jnp.int32),
        jax.ShapeDtypeStruct((TOP_K, n), jnp.float32),
    )
    topi, topv = pl.pallas_call(
        _router_kernel,
        grid=grid,
        in_specs=[
            pl.BlockSpec((BT, D_MODEL), lambda i: (i, 0)),
            pl.BlockSpec((NUM_EXPERTS, D_MODEL), lambda i: (0, 0)),
            pl.BlockSpec((NUM_EXPERTS, 1), lambda i: (0, 0)),
        ],
        out_specs=(
            pl.BlockSpec((TOP_K, BT), lambda i: (0, i)),
            pl.BlockSpec((TOP_K, BT), lambda i: (0, i)),
        ),
        out_shape=out_shape,
    )(x, W, b.reshape(NUM_EXPERTS, 1))
    return topi.T, topv.T
